# Initial kernel scaffold; baseline (speedup 1.0000x reference)
import jax
import jax.numpy as jnp
from jax.experimental import pallas as pl


def kernel(input):
    raise NotImplementedError("write your pallas kernel here")


# speedup vs baseline: 14.9071x; 14.9071x over previous
import functools

import jax
import jax.numpy as jnp
from jax import lax
from jax.experimental import pallas as pl
from jax.experimental.pallas import tpu as pltpu
from jax.experimental.pallas import tpu_sc as plsc

NUM_MAPS = 4
KMAX = 20
KMIN = 20
ALPHA = 0.7

L = 16
NC = 2
NS = 16
NW = NC * NS

N = 2304
NVR = N // L
ROWS = 12288
ROWS_PER_W = ROWS // NW
RPC = 16
NCH = ROWS_PER_W // RPC
CH_ELEMS = RPC * N

NB = 4
FOLD_STEPS = NVR // NB

NEG = -3.0e38


def _sort_desc(v):
  k, _ = plsc.sort_key_val(v, v, descending=True)
  return k


def _merge_halves(a_desc, b_desc):
  b_asc = lax.rev(b_desc, (0,))
  return jnp.maximum(a_desc, b_asc), jnp.minimum(a_desc, b_asc)


def _merge32(b1, b2, s_desc):
  t_hi, _ = _merge_halves(b2, s_desc)
  t_hi = _sort_desc(t_hi)
  u, w = _merge_halves(b1, t_hi)
  return _sort_desc(u), _sort_desc(w)


def _nth_of_sorted32(b2, i):
  lane = lax.iota(jnp.int32, L)
  return jnp.max(jnp.where(lane == i, b2, NEG))


def _top20_sum(b1, b2):
  lane = lax.iota(jnp.int32, L)
  return jnp.sum(b1) + jnp.sum(jnp.where(lane < KMAX - L, b2, jnp.float32(0.0)))


def _sc_body(x_hbm, out_hbm, buf0, buf1, cand_hi, cand_lo, obuf, sem0, sem1):
  wid = lax.axis_index("s") * NC + lax.axis_index("c")
  base_elem = wid * ROWS_PER_W * N
  lane = lax.iota(jnp.int32, L)

  def start_chunk(c, buf, sem):
    off = base_elem + c * CH_ELEMS
    pltpu.async_copy(x_hbm.at[pl.ds(off, CH_ELEMS)], buf, sem)

  def wait_chunk(buf, sem):
    pltpu.make_async_copy(x_hbm.at[pl.ds(0, CH_ELEMS)], buf, sem).wait()

  def process_row(buf, row):
    row_off = row * N

    def fold_body(i, carry):
      amax = list(carry[:NB])
      amin = list(carry[NB:])
      for j in range(NB):
        v = buf[pl.ds(row_off + (i * NB + j) * L, L)]
        amax[j] = jnp.maximum(amax[j], v)
        amin[j] = jnp.minimum(amin[j], v)
      return tuple(amax) + tuple(amin)

    init = tuple([jnp.full((L,), NEG, jnp.float32)] * NB) + tuple(
        [jnp.full((L,), -NEG, jnp.float32)] * NB)
    accs = lax.fori_loop(0, FOLD_STEPS, fold_body, init)
    amax = accs[:NB]
    amin = accs[NB:]

    def nth20(vregs):
      s = [_sort_desc(v) for v in vregs]
      hi, lo = _merge_halves(s[0], s[1])
      b1, b2 = _sort_desc(hi), _sort_desc(lo)
      for k in range(2, NB):
        b1, b2 = _merge32(b1, b2, s[k])
      return _nth_of_sorted32(b2, KMAX - L - 1)

    t_hi = nth20(amax)
    t_lo_neg = nth20([-v for v in amin])

    def filt_body(i, carry):
      c_hi, c_lo = carry
      for j in range(NB):
        v = buf[pl.ds(row_off + (i * NB + j) * L, L)]
        nv = -v
        mh = v >= t_hi
        ml = nv >= t_lo_neg
        plsc.store_compressed(cand_hi.at[pl.ds(c_hi, L)], v, mask=mh)
        plsc.store_compressed(cand_lo.at[pl.ds(c_lo, L)], nv, mask=ml)
        c_hi = c_hi + jnp.sum(mh.astype(jnp.int32))
        c_lo = c_lo + jnp.sum(ml.astype(jnp.int32))
      return c_hi, c_lo

    c_hi, c_lo = lax.fori_loop(
        0, FOLD_STEPS, filt_body, (jnp.int32(0), jnp.int32(0)))

    def topsum(cand, cnt):
      def mbody(i, carry):
        b1, b2 = carry
        v = cand[pl.ds(i * L, L)]
        v = jnp.where(lane < cnt - i * L, v, NEG)
        return _merge32(b1, b2, _sort_desc(v))

      b1 = jnp.full((L,), NEG, jnp.float32)
      b2 = jnp.full((L,), NEG, jnp.float32)
      nv = lax.shift_right_logical(cnt + (L - 1), 4)
      b1, b2 = lax.fori_loop(0, nv, mbody, (b1, b2))
      return _top20_sum(b1, b2)

    s_top = topsum(cand_hi, c_hi)
    s_bot = -topsum(cand_lo, c_lo)

    return (s_top * (1.0 / KMAX) + s_bot * (ALPHA / KMIN)) * jnp.float32(0.5)

  def process_chunk(buf, c):
    def rows_body(r, acc):
      res = process_row(buf, r)
      return jnp.where(lane == r, res, acc)

    acc = lax.fori_loop(
        0, RPC, rows_body, jnp.zeros((L,), jnp.float32))
    obuf[pl.ds(c * RPC, L)] = acc

  start_chunk(0, buf0, sem0)
  start_chunk(1, buf1, sem1)

  def pair_body(p, _):
    c0 = 2 * p

    wait_chunk(buf0, sem0)
    process_chunk(buf0, c0)

    @pl.when(p < NCH // 2 - 1)
    def _():
      start_chunk(c0 + 2, buf0, sem0)

    wait_chunk(buf1, sem1)
    process_chunk(buf1, c0 + 1)

    @pl.when(p < NCH // 2 - 1)
    def _():
      start_chunk(c0 + 3, buf1, sem1)

    return 0

  lax.fori_loop(0, NCH // 2, pair_body, 0)

  pltpu.sync_copy(obuf, out_hbm.at[pl.ds(wid * ROWS_PER_W, ROWS_PER_W)])


@jax.jit
def kernel(input):
  batch, ch, h, w = input.shape
  num_outputs = ch // NUM_MAPS
  x = input.reshape(batch * num_outputs, NUM_MAPS * h * w).reshape(-1)

  mesh = plsc.VectorSubcoreMesh(
      core_axis_name="c", subcore_axis_name="s",
      num_cores=NC, num_subcores=NS)
  run = functools.partial(
      pl.kernel,
      out_type=jax.ShapeDtypeStruct((ROWS,), jnp.float32),
      mesh=mesh,
      scratch_types=[
          pltpu.VMEM((CH_ELEMS,), jnp.float32),
          pltpu.VMEM((CH_ELEMS,), jnp.float32),
          pltpu.VMEM((N + L,), jnp.float32),
          pltpu.VMEM((N + L,), jnp.float32),
          pltpu.VMEM((ROWS_PER_W,), jnp.float32),
          pltpu.SemaphoreType.DMA,
          pltpu.SemaphoreType.DMA,
      ],
      compiler_params=pltpu.CompilerParams(needs_layout_passes=False),
  )(_sc_body)
  out = run(x)
  return out.reshape(batch, num_outputs)


# speedup vs baseline: 17.4400x; 1.1699x over previous
import functools

import jax
import jax.numpy as jnp
from jax import lax
from jax.experimental import pallas as pl
from jax.experimental.pallas import tpu as pltpu
from jax.experimental.pallas import tpu_sc as plsc

NUM_MAPS = 4
KMAX = 20
KMIN = 20
ALPHA = 0.7

L = 16
NC = 2
NS = 16
NW = NC * NS

N = 2304
NVR = N // L
ROWS = 12288
ROWS_PER_W = ROWS // NW
RPC = 16
NCH = ROWS_PER_W // RPC
CH_ELEMS = RPC * N

IL = 8
NB = 4
FOLD_STEPS = NVR // NB
CAP = N + L
SMERGE = 4

NEG = -3.0e38


def _sort_desc(v):
  k, _ = plsc.sort_key_val(v, v, descending=True)
  return k


def _merge_halves(a_desc, b_desc):
  b_asc = lax.rev(b_desc, (0,))
  return jnp.maximum(a_desc, b_asc), jnp.minimum(a_desc, b_asc)


def _merge32(b1, b2, s_desc):
  t_hi, _ = _merge_halves(b2, s_desc)
  t_hi = _sort_desc(t_hi)
  u, w = _merge_halves(b1, t_hi)
  return _sort_desc(u), _sort_desc(w)


def _sc_body(x_hbm, out_hbm, bufs, cand, obuf, sem0, sem1):
  wid = lax.axis_index("s") * NC + lax.axis_index("c")
  base_elem = wid * ROWS_PER_W * N
  lane = lax.iota(jnp.int32, L)

  def start_chunk(c, sel_static, sem):
    off = base_elem + c * CH_ELEMS
    pltpu.async_copy(
        x_hbm.at[pl.ds(off, CH_ELEMS)],
        bufs.at[pl.ds(sel_static * CH_ELEMS, CH_ELEMS)], sem)

  def wait_chunk(sel_static, sem):
    pltpu.make_async_copy(
        x_hbm.at[pl.ds(0, CH_ELEMS)],
        bufs.at[pl.ds(sel_static * CH_ELEMS, CH_ELEMS)], sem).wait()

  def fold_row(row_base):
    def fold_body(i, carry):
      amax = list(carry[:NB])
      amin = list(carry[NB:])
      for j in range(NB):
        v = bufs[pl.ds(row_base + (i * NB + j) * L, L)]
        amax[j] = jnp.maximum(amax[j], v)
        amin[j] = jnp.minimum(amin[j], v)
      return tuple(amax) + tuple(amin)

    init = tuple([jnp.full((L,), NEG, jnp.float32)] * NB) + tuple(
        [jnp.full((L,), -NEG, jnp.float32)] * NB)
    accs = lax.fori_loop(0, FOLD_STEPS, fold_body, init)
    return accs[:NB], accs[NB:]

  def nth20(vregs):
    s = [_sort_desc(v) for v in vregs]
    hi, lo = _merge_halves(s[0], s[1])
    b1, b2 = _sort_desc(hi), _sort_desc(lo)
    for k in range(2, NB):
      b1, b2 = _merge32(b1, b2, s[k])
    return jnp.max(jnp.where(lane == (KMAX - L - 1), b2, NEG))

  def process_group(buf_base, grow0, lane0):
    rbase = [buf_base + (grow0 + r) * N for r in range(IL)]

    th_splat = []
    tl_splat = []
    for r in range(IL):
      amax, amin = fold_row(rbase[r])
      t_hi = nth20(amax)
      t_lo = -nth20([-v for v in amin])
      th_splat.append(jnp.full((L,), t_hi, jnp.float32))
      tl_splat.append(jnp.full((L,), t_lo, jnp.float32))

    def filt_body(i, carry):
      cnt = list(carry)
      for r in range(IL):
        v = bufs[pl.ds(rbase[r] + i * L, L)]
        m = (v >= th_splat[r]) | (v <= tl_splat[r])
        plsc.store_compressed(cand.at[pl.ds(r * CAP + cnt[r], L)], v, mask=m)
        cnt[r] = cnt[r] + jnp.sum(m.astype(jnp.int32))
      return tuple(cnt)

    cnts = lax.fori_loop(0, NVR, filt_body, (jnp.int32(0),) * IL)

    def masked_cand(r, i, negate):
      v = cand[pl.ds(r * CAP + i * L, L)]
      if negate:
        v = -v
      return jnp.where(lane < cnts[r] - i * L, v, NEG)

    def static_merges(negate):
      b1 = [jnp.full((L,), NEG, jnp.float32) for _ in range(IL)]
      b2 = [jnp.full((L,), NEG, jnp.float32) for _ in range(IL)]
      for i in range(SMERGE):
        for r in range(IL):
          b1[r], b2[r] = _merge32(
              b1[r], b2[r], _sort_desc(masked_cand(r, i, negate)))
      return b1, b2

    def dyn_tail(b1, b2, r, negate):
      nv = lax.shift_right_logical(cnts[r] + (L - 1), 4)

      def mbody(i, carry):
        return _merge32(*carry, _sort_desc(masked_cand(r, i, negate)))

      return lax.fori_loop(SMERGE, nv, mbody, (b1, b2))

    def sum20(b1, b2):
      return jnp.sum(b1) + jnp.sum(
          jnp.where(lane < KMAX - L, b2, jnp.float32(0.0)))

    h1, h2 = static_merges(False)
    l1, l2 = static_merges(True)
    acc = jnp.zeros((L,), jnp.float32)
    for r in range(IL):
      hb1, hb2 = dyn_tail(h1[r], h2[r], r, False)
      lb1, lb2 = dyn_tail(l1[r], l2[r], r, True)
      s_top = sum20(hb1, hb2)
      s_bot = -sum20(lb1, lb2)
      res = (s_top * (1.0 / KMAX) + s_bot * (ALPHA / KMIN)) * jnp.float32(0.5)
      acc = jnp.where(lane == lane0 + r, res, acc)
    return acc

  start_chunk(0, 0, sem0)
  start_chunk(1, 1, sem1)

  def chunk_body(c, _):
    sel = jnp.bitwise_and(c, 1)
    buf_base = sel * CH_ELEMS

    @pl.when(sel == 0)
    def _():
      wait_chunk(0, sem0)

    @pl.when(sel == 1)
    def _():
      wait_chunk(1, sem1)

    def group_body(g, acc):
      return acc + process_group(buf_base, g * IL, g * IL)

    acc = lax.fori_loop(
        0, RPC // IL, group_body, jnp.zeros((L,), jnp.float32))
    obuf[pl.ds(c * RPC, L)] = acc

    @pl.when(c + 2 < NCH)
    def _():
      @pl.when(sel == 0)
      def _():
        start_chunk(c + 2, 0, sem0)

      @pl.when(sel == 1)
      def _():
        start_chunk(c + 2, 1, sem1)

    return 0

  lax.fori_loop(0, NCH, chunk_body, 0)

  pltpu.sync_copy(obuf, out_hbm.at[pl.ds(wid * ROWS_PER_W, ROWS_PER_W)])


@jax.jit
def kernel(input):
  batch, ch, h, w = input.shape
  num_outputs = ch // NUM_MAPS
  x = input.reshape(batch * num_outputs, NUM_MAPS * h * w).reshape(-1)

  mesh = plsc.VectorSubcoreMesh(
      core_axis_name="c", subcore_axis_name="s",
      num_cores=NC, num_subcores=NS)
  run = functools.partial(
      pl.kernel,
      out_type=jax.ShapeDtypeStruct((ROWS,), jnp.float32),
      mesh=mesh,
      scratch_types=[
          pltpu.VMEM((2 * CH_ELEMS,), jnp.float32),
          pltpu.VMEM((IL * CAP,), jnp.float32),
          pltpu.VMEM((ROWS_PER_W,), jnp.float32),
          pltpu.SemaphoreType.DMA,
          pltpu.SemaphoreType.DMA,
      ],
      compiler_params=pltpu.CompilerParams(needs_layout_passes=False),
  )(_sc_body)
  out = run(x)
  return out.reshape(batch, num_outputs)


# speedup vs baseline: 17.4588x; 1.0011x over previous
import functools

import jax
import jax.numpy as jnp
from jax import lax
from jax.experimental import pallas as pl
from jax.experimental.pallas import tpu as pltpu
from jax.experimental.pallas import tpu_sc as plsc

NUM_MAPS = 4
KMAX = 20
KMIN = 20
ALPHA = 0.7

L = 16
NC = 2
NS = 16
NW = NC * NS

N = 2304
NVR = N // L
ROWS = 12288
ROWS_PER_W = ROWS // NW
RPC = 16
NCH = ROWS_PER_W // RPC
CH_ELEMS = RPC * N

IL = 8
NB = 4
FOLD_STEPS = NVR // NB
CAP = N + L
SMERGE = 4

NEG = -3.0e38


def _sort_desc(v):
  k, _ = plsc.sort_key_val(v, v, descending=True)
  return k


def _merge_halves(a_desc, b_desc):
  b_asc = lax.rev(b_desc, (0,))
  return jnp.maximum(a_desc, b_asc), jnp.minimum(a_desc, b_asc)


def _merge32(b1, b2, s_desc):
  t_hi, _ = _merge_halves(b2, s_desc)
  t_hi = _sort_desc(t_hi)
  u, w = _merge_halves(b1, t_hi)
  return _sort_desc(u), _sort_desc(w)


def _sc_body(x_hbm, out_hbm, bufs, cand, obuf, sem0, sem1):
  wid = lax.axis_index("s") * NC + lax.axis_index("c")
  base_elem = wid * ROWS_PER_W * N
  lane = lax.iota(jnp.int32, L)

  def start_chunk(c, sel_static, sem):
    off = base_elem + c * CH_ELEMS
    pltpu.async_copy(
        x_hbm.at[pl.ds(off, CH_ELEMS)],
        bufs.at[pl.ds(sel_static * CH_ELEMS, CH_ELEMS)], sem)

  def wait_chunk(sel_static, sem):
    pltpu.make_async_copy(
        x_hbm.at[pl.ds(0, CH_ELEMS)],
        bufs.at[pl.ds(sel_static * CH_ELEMS, CH_ELEMS)], sem).wait()

  def fold_row(row_base):
    def fold_body(i, carry):
      amax = list(carry[:NB])
      amin = list(carry[NB:])
      for j in range(NB):
        v = bufs[pl.ds(row_base + (i * NB + j) * L, L)]
        amax[j] = jnp.maximum(amax[j], v)
        amin[j] = jnp.minimum(amin[j], v)
      return tuple(amax) + tuple(amin)

    init = tuple([jnp.full((L,), NEG, jnp.float32)] * NB) + tuple(
        [jnp.full((L,), -NEG, jnp.float32)] * NB)
    accs = lax.fori_loop(0, FOLD_STEPS, fold_body, init)
    return accs[:NB], accs[NB:]

  def nth20(vregs):
    s = [_sort_desc(v) for v in vregs]
    hi, lo = _merge_halves(s[0], s[1])
    b1, b2 = _sort_desc(hi), _sort_desc(lo)
    for k in range(2, NB):
      b1, b2 = _merge32(b1, b2, s[k])
    return jnp.max(jnp.where(lane == (KMAX - L - 1), b2, NEG))

  def process_group(buf_base, grow0, lane0):
    rbase = [buf_base + (grow0 + r) * N for r in range(IL)]

    t_his = []
    t_los = []
    for r in range(IL):
      amax, amin = fold_row(rbase[r])
      t_his.append(nth20(amax))
      t_los.append(-nth20([-v for v in amin]))

    def filt_body(i, carry):
      cnt = list(carry[:IL])
      ths = carry[IL:2 * IL]
      tls = carry[2 * IL:]
      for r in range(IL):
        v = bufs[pl.ds(rbase[r] + i * L, L)]
        m = (v >= ths[r]) | (v <= tls[r])
        plsc.store_compressed(cand.at[pl.ds(r * CAP + cnt[r], L)], v, mask=m)
        cnt[r] = cnt[r] + jnp.sum(m.astype(jnp.int32))
      return tuple(cnt) + tuple(ths) + tuple(tls)

    carry0 = ((jnp.int32(0),) * IL) + tuple(t_his) + tuple(t_los)
    cnts = lax.fori_loop(0, NVR, filt_body, carry0)[:IL]

    def masked_cand(r, i, negate):
      v = cand[pl.ds(r * CAP + i * L, L)]
      if negate:
        v = -v
      return jnp.where(lane < cnts[r] - i * L, v, NEG)

    def static_merges(negate):
      b1 = [jnp.full((L,), NEG, jnp.float32) for _ in range(IL)]
      b2 = [jnp.full((L,), NEG, jnp.float32) for _ in range(IL)]
      for i in range(SMERGE):
        for r in range(IL):
          b1[r], b2[r] = _merge32(
              b1[r], b2[r], _sort_desc(masked_cand(r, i, negate)))
      return b1, b2

    def dyn_tail(b1, b2, r, negate):
      nv = lax.shift_right_logical(cnts[r] + (L - 1), 4)

      def mbody(i, carry):
        return _merge32(*carry, _sort_desc(masked_cand(r, i, negate)))

      return lax.fori_loop(SMERGE, nv, mbody, (b1, b2))

    def sum20(b1, b2):
      return jnp.sum(b1) + jnp.sum(
          jnp.where(lane < KMAX - L, b2, jnp.float32(0.0)))

    h1, h2 = static_merges(False)
    l1, l2 = static_merges(True)
    acc = jnp.zeros((L,), jnp.float32)
    for r in range(IL):
      hb1, hb2 = dyn_tail(h1[r], h2[r], r, False)
      lb1, lb2 = dyn_tail(l1[r], l2[r], r, True)
      s_top = sum20(hb1, hb2)
      s_bot = -sum20(lb1, lb2)
      res = (s_top * (1.0 / KMAX) + s_bot * (ALPHA / KMIN)) * jnp.float32(0.5)
      acc = jnp.where(lane == lane0 + r, res, acc)
    return acc

  start_chunk(0, 0, sem0)
  start_chunk(1, 1, sem1)

  def chunk_body(c, _):
    sel = jnp.bitwise_and(c, 1)
    buf_base = sel * CH_ELEMS

    @pl.when(sel == 0)
    def _():
      wait_chunk(0, sem0)

    @pl.when(sel == 1)
    def _():
      wait_chunk(1, sem1)

    def group_body(g, acc):
      return acc + process_group(buf_base, g * IL, g * IL)

    acc = lax.fori_loop(
        0, RPC // IL, group_body, jnp.zeros((L,), jnp.float32))
    obuf[pl.ds(c * RPC, L)] = acc

    @pl.when(c + 2 < NCH)
    def _():
      @pl.when(sel == 0)
      def _():
        start_chunk(c + 2, 0, sem0)

      @pl.when(sel == 1)
      def _():
        start_chunk(c + 2, 1, sem1)

    return 0

  lax.fori_loop(0, NCH, chunk_body, 0)

  pltpu.sync_copy(obuf, out_hbm.at[pl.ds(wid * ROWS_PER_W, ROWS_PER_W)])


@jax.jit
def kernel(input):
  batch, ch, h, w = input.shape
  num_outputs = ch // NUM_MAPS
  x = input.reshape(batch * num_outputs, NUM_MAPS * h * w).reshape(-1)

  mesh = plsc.VectorSubcoreMesh(
      core_axis_name="c", subcore_axis_name="s",
      num_cores=NC, num_subcores=NS)
  run = functools.partial(
      pl.kernel,
      out_type=jax.ShapeDtypeStruct((ROWS,), jnp.float32),
      mesh=mesh,
      scratch_types=[
          pltpu.VMEM((2 * CH_ELEMS,), jnp.float32),
          pltpu.VMEM((IL * CAP,), jnp.float32),
          pltpu.VMEM((ROWS_PER_W,), jnp.float32),
          pltpu.SemaphoreType.DMA,
          pltpu.SemaphoreType.DMA,
      ],
      compiler_params=pltpu.CompilerParams(needs_layout_passes=False),
  )(_sc_body)
  out = run(x)
  return out.reshape(batch, num_outputs)
